# baseline (speedup 1.0000x reference)
import functools

import jax
import jax.numpy as jnp
import numpy as np
from jax import lax
from jax.experimental import pallas as pl
from jax.experimental.pallas import tpu as pltpu
from jax.experimental.pallas import tpu_sc as plsc

B = 16384
EMB = 64
NC = 2
NS = 16
NW = NC * NS
BPW = B // NW
CHUNK = 256

N_ROWS = 100000
PREP_BLK = 16384
TC_BLK = 4096

HI_MASK = np.uint32(0xFFFF0000)


def _transpose_mxu(x_ref, eye):
    return lax.dot_general(x_ref[...].astype(jnp.bfloat16), eye,
                           (((0,), (0,)), ((), ())),
                           preferred_element_type=jnp.float32)


def _prep_body(g_ref, m_ref, eye_ref, o_ref):
    eye = eye_ref[...]
    gt = _transpose_mxu(g_ref, eye)
    mt = _transpose_mxu(m_ref, eye)
    hi = lax.bitcast_convert_type(gt, jnp.uint32)
    lo = lax.bitcast_convert_type(mt, jnp.uint32) >> 16
    o_ref[...] = lax.bitcast_convert_type(hi | lo, jnp.float32)


def _tc_prep(gT, mT):
    grid = (N_ROWS + PREP_BLK - 1) // PREP_BLK
    eye = jnp.eye(EMB, dtype=jnp.bfloat16)
    return pl.pallas_call(
        _prep_body,
        grid=(grid,),
        in_specs=[pl.BlockSpec((EMB, PREP_BLK), lambda i: (0, i)),
                  pl.BlockSpec((EMB, PREP_BLK), lambda i: (0, i)),
                  pl.BlockSpec((EMB, EMB), lambda i: (0, 0))],
        out_specs=pl.BlockSpec((PREP_BLK, EMB), lambda i: (i, 0)),
        out_shape=jax.ShapeDtypeStruct((N_ROWS, EMB), jnp.float32),
    )(gT, mT, eye)


def _sc_gather(idxs, T):
    n = idxs.shape[0]
    bpw = n // NW
    nchunk = max(1, bpw // CHUNK)
    chunk = bpw // nchunk
    mesh = plsc.VectorSubcoreMesh(core_axis_name="c", subcore_axis_name="s")
    row_t = jax.ShapeDtypeStruct((n, EMB), jnp.float32)
    buf_t = pltpu.VMEM((chunk, EMB), jnp.float32)

    @functools.partial(
        pl.kernel,
        out_type=row_t,
        mesh=mesh,
        scratch_types=[
            pltpu.VMEM((bpw,), jnp.int32),
            buf_t, buf_t,
            pltpu.SemaphoreType.DMA,
            pltpu.SemaphoreType.DMA,
        ],
    )
    def gather_kernel(i_hbm, T_hbm, o_hbm, idx, b0, b1, s0, s1):
        wid = lax.axis_index("s") * NC + lax.axis_index("c")
        base = wid * bpw
        pltpu.sync_copy(i_hbm.at[pl.ds(base, bpw)], idx)

        def fire(c, buf, sem):
            off = c * chunk

            @pl.loop(0, chunk, step=16)
            def _(r):
                vec = idx[pl.ds(off + r, 16)]
                for j in range(16):
                    pltpu.async_copy(T_hbm.at[pl.ds(vec[j], 1)],
                                     buf.at[pl.ds(r + j, 1)], sem)

        def drain_store(c, buf, sem):
            pltpu.make_async_copy(T_hbm.at[pl.ds(0, chunk)], buf, sem).wait()
            pltpu.sync_copy(buf, o_hbm.at[pl.ds(base + c * chunk, chunk)])

        fire(0, b0, s0)
        for c in range(nchunk):
            if c + 1 < nchunk:
                fire(c + 1, (b0, b1)[(c + 1) % 2], (s0, s1)[(c + 1) % 2])
            drain_store(c, (b0, b1)[c % 2], (s0, s1)[c % 2])

    return gather_kernel(idxs, T)


def _unpack(x):
    w = lax.bitcast_convert_type(x, jnp.uint32)
    hi = lax.bitcast_convert_type(w & HI_MASK, jnp.float32)
    lo = lax.bitcast_convert_type(w << 16, jnp.float32)
    return hi, lo


def _bf16(x):
    return x.astype(jnp.bfloat16)


def _dotT(w, x):
    return lax.dot_general(w, x, (((1,), (1,)), ((), ())),
                           preferred_element_type=jnp.float32)


def _tc_body(u_ref, v_ref, w1u_ref, w1i_ref, eye_ref, b1_ref, w2_ref,
             b2_ref, w3_ref, b3_ref, o_ref):
    gmf_u, mlp_u = _unpack(u_ref[...])
    gmf_v, mlp_v = _unpack(v_ref[...])
    h1 = _dotT(w1u_ref[...], _bf16(mlp_u)) + _dotT(w1i_ref[...], _bf16(mlp_v))
    h1 = jnp.maximum(h1 + b1_ref[...], 0.0)
    h2 = jnp.dot(w2_ref[...], _bf16(h1), preferred_element_type=jnp.float32)
    h2 = jnp.maximum(h2 + b2_ref[...], 0.0)
    mlp = jnp.dot(w3_ref[...], _bf16(h2),
                  preferred_element_type=jnp.float32)
    pT = _dotT(eye_ref[...], _bf16(gmf_u * gmf_v))
    gmf = jnp.sum(pT, axis=0, keepdims=True)
    o_ref[...] = jax.nn.sigmoid(gmf + mlp + b3_ref[...])[0]


def _tc_dense(gu, gi, W1, b1, W2, b2, W3, b3):
    w1u = W1[:, :EMB].astype(jnp.bfloat16)
    w1i = W1[:, EMB:].astype(jnp.bfloat16)
    w2b = W2.astype(jnp.bfloat16)
    w3b = W3.astype(jnp.bfloat16)
    eye = jnp.eye(EMB, dtype=jnp.bfloat16)
    b1c = b1.reshape(-1, 1)
    b2c = b2.reshape(-1, 1)
    b3c = b3.reshape(1, 1)

    n = gu.shape[0]
    grid = n // TC_BLK
    blk = lambda: pl.BlockSpec((TC_BLK, EMB), lambda i: (i, 0))
    full = lambda a: pl.BlockSpec(a.shape, lambda i: (0,) * a.ndim)
    out = pl.pallas_call(
        _tc_body,
        grid=(grid,),
        in_specs=[blk(), blk(),
                  full(w1u), full(w1i), full(eye), full(b1c),
                  full(w2b), full(b2c), full(w3b), full(b3c)],
        out_specs=pl.BlockSpec((TC_BLK,), lambda i: (i,)),
        out_shape=jax.ShapeDtypeStruct((n,), jnp.float32),
    )(gu, gi, w1u, w1i, eye, b1c, w2b, b2c, w3b, b3c)
    return out


def kernel(users, items, gmf_user_W, gmf_item_W, mlp_user_W, mlp_item_W,
           W1, b1, W2, b2, W3, b3):
    users = users.astype(jnp.int32)
    items = items.astype(jnp.int32)
    H = B // 2
    U = _tc_prep(gmf_user_W.T, mlp_user_W.T)
    gu = _sc_gather(users, U)
    V = _tc_prep(gmf_item_W.T, mlp_item_W.T)
    gi_a = _sc_gather(items[:H], V)
    out_a = _tc_dense(gu[:H], gi_a, W1, b1, W2, b2, W3, b3)
    gi_b = _sc_gather(items[H:], V)
    out_b = _tc_dense(gu[H:], gi_b, W1, b1, W2, b2, W3, b3)
    return jnp.concatenate([out_a, out_b])


# speedup vs baseline: 1.0817x; 1.0817x over previous
import functools

import jax
import jax.numpy as jnp
import numpy as np
from jax import lax
from jax.experimental import pallas as pl
from jax.experimental.pallas import tpu as pltpu
from jax.experimental.pallas import tpu_sc as plsc

B = 16384
EMB = 64
NC = 2
NS = 16
NW = NC * NS
BPW = B // NW
CHUNK = 256
NCHUNK = BPW // CHUNK

N_ROWS = 100000
PREP_BLK = 25088
TC_BLK = 4096

HI_MASK = np.uint32(0xFFFF0000)


def _transpose_mxu(x_ref, eye):
    return lax.dot_general(x_ref[...].astype(jnp.bfloat16), eye,
                           (((0,), (0,)), ((), ())),
                           preferred_element_type=jnp.float32)


def _prep_body(g_ref, m_ref, eye_ref, o_ref):
    eye = eye_ref[...]
    gt = _transpose_mxu(g_ref, eye)
    mt = _transpose_mxu(m_ref, eye)
    hi = lax.bitcast_convert_type(gt, jnp.uint32)
    lo = lax.bitcast_convert_type(mt, jnp.uint32) >> 16
    o_ref[...] = lax.bitcast_convert_type(hi | lo, jnp.float32)


def _tc_prep(gT, mT):
    grid = (N_ROWS + PREP_BLK - 1) // PREP_BLK
    eye = jnp.eye(EMB, dtype=jnp.bfloat16)
    return pl.pallas_call(
        _prep_body,
        grid=(grid,),
        in_specs=[pl.BlockSpec((EMB, PREP_BLK), lambda i: (0, i)),
                  pl.BlockSpec((EMB, PREP_BLK), lambda i: (0, i)),
                  pl.BlockSpec((EMB, EMB), lambda i: (0, 0))],
        out_specs=pl.BlockSpec((PREP_BLK, EMB), lambda i: (i, 0)),
        out_shape=jax.ShapeDtypeStruct((N_ROWS, EMB), jnp.float32),
    )(gT, mT, eye)


def _sc_gather(idxs, T):
    mesh = plsc.VectorSubcoreMesh(core_axis_name="c", subcore_axis_name="s")
    row_t = jax.ShapeDtypeStruct((B, EMB), jnp.float32)
    buf_t = pltpu.VMEM((CHUNK, EMB), jnp.float32)

    @functools.partial(
        pl.kernel,
        out_type=row_t,
        mesh=mesh,
        scratch_types=[
            pltpu.VMEM((BPW,), jnp.int32),
            buf_t, buf_t,
            pltpu.SemaphoreType.DMA,
            pltpu.SemaphoreType.DMA,
        ],
    )
    def gather_kernel(i_hbm, T_hbm, o_hbm, idx, b0, b1, s0, s1):
        wid = lax.axis_index("s") * NC + lax.axis_index("c")
        base = wid * BPW
        pltpu.sync_copy(i_hbm.at[pl.ds(base, BPW)], idx)

        def fire(c, buf, sem):
            off = c * CHUNK

            @pl.loop(0, CHUNK, step=16)
            def _(r):
                vec = idx[pl.ds(off + r, 16)]
                for j in range(16):
                    pltpu.async_copy(T_hbm.at[pl.ds(vec[j], 1)],
                                     buf.at[pl.ds(r + j, 1)], sem)

        def drain_store(c, buf, sem):
            pltpu.make_async_copy(T_hbm.at[pl.ds(0, CHUNK)], buf, sem).wait()
            pltpu.sync_copy(buf, o_hbm.at[pl.ds(base + c * CHUNK, CHUNK)])

        fire(0, b0, s0)
        for c in range(NCHUNK):
            if c + 1 < NCHUNK:
                fire(c + 1, (b0, b1)[(c + 1) % 2], (s0, s1)[(c + 1) % 2])
            drain_store(c, (b0, b1)[c % 2], (s0, s1)[c % 2])

    return gather_kernel(idxs, T)


def _unpack(x):
    w = lax.bitcast_convert_type(x, jnp.uint32)
    hi = lax.bitcast_convert_type(w & HI_MASK, jnp.float32)
    lo = lax.bitcast_convert_type(w << 16, jnp.float32)
    return hi, lo


def _bf16(x):
    return x.astype(jnp.bfloat16)


def _dotT(w, x):
    return lax.dot_general(w, x, (((1,), (1,)), ((), ())),
                           preferred_element_type=jnp.float32)


def _tc_body(u_ref, v_ref, w1u_ref, w1i_ref, eye_ref, b1_ref, w2_ref,
             b2_ref, w3_ref, b3_ref, o_ref):
    gmf_u, mlp_u = _unpack(u_ref[...])
    gmf_v, mlp_v = _unpack(v_ref[...])
    h1 = _dotT(w1u_ref[...], _bf16(mlp_u)) + _dotT(w1i_ref[...], _bf16(mlp_v))
    h1 = jnp.maximum(h1 + b1_ref[...], 0.0)
    h2 = jnp.dot(w2_ref[...], _bf16(h1), preferred_element_type=jnp.float32)
    h2 = jnp.maximum(h2 + b2_ref[...], 0.0)
    mlp = jnp.dot(w3_ref[...], _bf16(h2),
                  preferred_element_type=jnp.float32)
    pT = _dotT(eye_ref[...], _bf16(gmf_u * gmf_v))
    gmf = jnp.sum(pT, axis=0, keepdims=True)
    o_ref[...] = jax.nn.sigmoid(gmf + mlp + b3_ref[...])[0]


def _tc_dense(gu, gi, W1, b1, W2, b2, W3, b3):
    w1u = W1[:, :EMB].astype(jnp.bfloat16)
    w1i = W1[:, EMB:].astype(jnp.bfloat16)
    w2b = W2.astype(jnp.bfloat16)
    w3b = W3.astype(jnp.bfloat16)
    eye = jnp.eye(EMB, dtype=jnp.bfloat16)
    b1c = b1.reshape(-1, 1)
    b2c = b2.reshape(-1, 1)
    b3c = b3.reshape(1, 1)

    grid = B // TC_BLK
    blk = lambda: pl.BlockSpec((TC_BLK, EMB), lambda i: (i, 0))
    full = lambda a: pl.BlockSpec(a.shape, lambda i: (0,) * a.ndim)
    out = pl.pallas_call(
        _tc_body,
        grid=(grid,),
        in_specs=[blk(), blk(),
                  full(w1u), full(w1i), full(eye), full(b1c),
                  full(w2b), full(b2c), full(w3b), full(b3c)],
        out_specs=pl.BlockSpec((TC_BLK,), lambda i: (i,)),
        out_shape=jax.ShapeDtypeStruct((B,), jnp.float32),
    )(gu, gi, w1u, w1i, eye, b1c, w2b, b2c, w3b, b3c)
    return out


def kernel(users, items, gmf_user_W, gmf_item_W, mlp_user_W, mlp_item_W,
           W1, b1, W2, b2, W3, b3):
    users = users.astype(jnp.int32)
    items = items.astype(jnp.int32)
    U = _tc_prep(gmf_user_W.T, mlp_user_W.T)
    gu = _sc_gather(users, U)
    V = _tc_prep(gmf_item_W.T, mlp_item_W.T)
    gi = _sc_gather(items, V)
    return _tc_dense(gu, gi, W1, b1, W2, b2, W3, b3)


# speedup vs baseline: 1.0858x; 1.0038x over previous
import functools

import jax
import jax.numpy as jnp
import numpy as np
from jax import lax
from jax.experimental import pallas as pl
from jax.experimental.pallas import tpu as pltpu
from jax.experimental.pallas import tpu_sc as plsc

B = 16384
EMB = 64
NC = 2
NS = 16
NW = NC * NS
BPW = B // NW
CHUNK = 128
NCHUNK = BPW // CHUNK

N_ROWS = 100000
PREP_BLK = 16384
TC_BLK = 4096

HI_MASK = np.uint32(0xFFFF0000)


def _transpose_mxu(x_ref, eye):
    return lax.dot_general(x_ref[...].astype(jnp.bfloat16), eye,
                           (((0,), (0,)), ((), ())),
                           preferred_element_type=jnp.float32)


def _prep_body(g_ref, m_ref, eye_ref, o_ref):
    eye = eye_ref[...]
    gt = _transpose_mxu(g_ref, eye)
    mt = _transpose_mxu(m_ref, eye)
    hi = lax.bitcast_convert_type(gt, jnp.uint32)
    lo = lax.bitcast_convert_type(mt, jnp.uint32) >> 16
    o_ref[...] = lax.bitcast_convert_type(hi | lo, jnp.float32)


def _tc_prep(gT, mT):
    grid = (N_ROWS + PREP_BLK - 1) // PREP_BLK
    eye = jnp.eye(EMB, dtype=jnp.bfloat16)
    return pl.pallas_call(
        _prep_body,
        grid=(grid,),
        in_specs=[pl.BlockSpec((EMB, PREP_BLK), lambda i: (0, i)),
                  pl.BlockSpec((EMB, PREP_BLK), lambda i: (0, i)),
                  pl.BlockSpec((EMB, EMB), lambda i: (0, 0))],
        out_specs=pl.BlockSpec((PREP_BLK, EMB), lambda i: (i, 0)),
        out_shape=jax.ShapeDtypeStruct((N_ROWS, EMB), jnp.float32),
    )(gT, mT, eye)


def _sc_gather(idxs, T):
    mesh = plsc.VectorSubcoreMesh(core_axis_name="c", subcore_axis_name="s")
    row_t = jax.ShapeDtypeStruct((B, EMB), jnp.float32)
    buf_t = pltpu.VMEM((CHUNK, EMB), jnp.float32)

    @functools.partial(
        pl.kernel,
        out_type=row_t,
        mesh=mesh,
        scratch_types=[
            pltpu.VMEM((BPW,), jnp.int32),
            buf_t, buf_t,
            pltpu.SemaphoreType.DMA,
            pltpu.SemaphoreType.DMA,
        ],
    )
    def gather_kernel(i_hbm, T_hbm, o_hbm, idx, b0, b1, s0, s1):
        wid = lax.axis_index("s") * NC + lax.axis_index("c")
        base = wid * BPW
        pltpu.sync_copy(i_hbm.at[pl.ds(base, BPW)], idx)

        def fire(c, buf, sem):
            off = c * CHUNK

            @pl.loop(0, CHUNK, step=16)
            def _(r):
                vec = idx[pl.ds(off + r, 16)]
                for j in range(16):
                    pltpu.async_copy(T_hbm.at[pl.ds(vec[j], 1)],
                                     buf.at[pl.ds(r + j, 1)], sem)

        def drain_store(c, buf, sem):
            pltpu.make_async_copy(T_hbm.at[pl.ds(0, CHUNK)], buf, sem).wait()
            pltpu.sync_copy(buf, o_hbm.at[pl.ds(base + c * CHUNK, CHUNK)])

        fire(0, b0, s0)
        for c in range(NCHUNK):
            if c + 1 < NCHUNK:
                fire(c + 1, (b0, b1)[(c + 1) % 2], (s0, s1)[(c + 1) % 2])
            drain_store(c, (b0, b1)[c % 2], (s0, s1)[c % 2])

    return gather_kernel(idxs, T)


def _unpack(x):
    w = lax.bitcast_convert_type(x, jnp.uint32)
    hi = lax.bitcast_convert_type(w & HI_MASK, jnp.float32)
    lo = lax.bitcast_convert_type(w << 16, jnp.float32)
    return hi, lo


def _bf16(x):
    return x.astype(jnp.bfloat16)


def _dotT(w, x):
    return lax.dot_general(w, x, (((1,), (1,)), ((), ())),
                           preferred_element_type=jnp.float32)


def _tc_body(u_ref, v_ref, w1u_ref, w1i_ref, eye_ref, b1_ref, w2_ref,
             b2_ref, w3_ref, b3_ref, o_ref):
    gmf_u, mlp_u = _unpack(u_ref[...])
    gmf_v, mlp_v = _unpack(v_ref[...])
    h1 = _dotT(w1u_ref[...], _bf16(mlp_u)) + _dotT(w1i_ref[...], _bf16(mlp_v))
    h1 = jnp.maximum(h1 + b1_ref[...], 0.0)
    h2 = jnp.dot(w2_ref[...], _bf16(h1), preferred_element_type=jnp.float32)
    h2 = jnp.maximum(h2 + b2_ref[...], 0.0)
    mlp = jnp.dot(w3_ref[...], _bf16(h2),
                  preferred_element_type=jnp.float32)
    pT = _dotT(eye_ref[...], _bf16(gmf_u * gmf_v))
    gmf = jnp.sum(pT, axis=0, keepdims=True)
    o_ref[...] = jax.nn.sigmoid(gmf + mlp + b3_ref[...])[0]


def _tc_dense(gu, gi, W1, b1, W2, b2, W3, b3):
    w1u = W1[:, :EMB].astype(jnp.bfloat16)
    w1i = W1[:, EMB:].astype(jnp.bfloat16)
    w2b = W2.astype(jnp.bfloat16)
    w3b = W3.astype(jnp.bfloat16)
    eye = jnp.eye(EMB, dtype=jnp.bfloat16)
    b1c = b1.reshape(-1, 1)
    b2c = b2.reshape(-1, 1)
    b3c = b3.reshape(1, 1)

    grid = B // TC_BLK
    blk = lambda: pl.BlockSpec((TC_BLK, EMB), lambda i: (i, 0))
    full = lambda a: pl.BlockSpec(a.shape, lambda i: (0,) * a.ndim)
    out = pl.pallas_call(
        _tc_body,
        grid=(grid,),
        in_specs=[blk(), blk(),
                  full(w1u), full(w1i), full(eye), full(b1c),
                  full(w2b), full(b2c), full(w3b), full(b3c)],
        out_specs=pl.BlockSpec((TC_BLK,), lambda i: (i,)),
        out_shape=jax.ShapeDtypeStruct((B,), jnp.float32),
    )(gu, gi, w1u, w1i, eye, b1c, w2b, b2c, w3b, b3c)
    return out


def kernel(users, items, gmf_user_W, gmf_item_W, mlp_user_W, mlp_item_W,
           W1, b1, W2, b2, W3, b3):
    users = users.astype(jnp.int32)
    items = items.astype(jnp.int32)
    U = _tc_prep(gmf_user_W.T, mlp_user_W.T)
    gu = _sc_gather(users, U)
    V = _tc_prep(gmf_item_W.T, mlp_item_W.T)
    gi = _sc_gather(items, V)
    return _tc_dense(gu, gi, W1, b1, W2, b2, W3, b3)


# speedup vs baseline: 1.0869x; 1.0009x over previous
import functools

import jax
import jax.numpy as jnp
import numpy as np
from jax import lax
from jax.experimental import pallas as pl
from jax.experimental.pallas import tpu as pltpu
from jax.experimental.pallas import tpu_sc as plsc

B = 16384
EMB = 64
NC = 2
NS = 16
NW = NC * NS
BPW = B // NW
CHUNK = 128
NCHUNK = BPW // CHUNK

N_ROWS = 100000
PREP_BLK = 16384
TC_BLK = 8192

HI_MASK = np.uint32(0xFFFF0000)


def _transpose_mxu(x_ref, eye):
    return lax.dot_general(x_ref[...].astype(jnp.bfloat16), eye,
                           (((0,), (0,)), ((), ())),
                           preferred_element_type=jnp.float32)


def _prep_body(g_ref, m_ref, eye_ref, o_ref):
    eye = eye_ref[...]
    gt = _transpose_mxu(g_ref, eye)
    mt = _transpose_mxu(m_ref, eye)
    hi = lax.bitcast_convert_type(gt, jnp.uint32)
    lo = lax.bitcast_convert_type(mt, jnp.uint32) >> 16
    o_ref[...] = lax.bitcast_convert_type(hi | lo, jnp.float32)


def _tc_prep(gT, mT):
    grid = (N_ROWS + PREP_BLK - 1) // PREP_BLK
    eye = jnp.eye(EMB, dtype=jnp.bfloat16)
    return pl.pallas_call(
        _prep_body,
        grid=(grid,),
        in_specs=[pl.BlockSpec((EMB, PREP_BLK), lambda i: (0, i)),
                  pl.BlockSpec((EMB, PREP_BLK), lambda i: (0, i)),
                  pl.BlockSpec((EMB, EMB), lambda i: (0, 0))],
        out_specs=pl.BlockSpec((PREP_BLK, EMB), lambda i: (i, 0)),
        out_shape=jax.ShapeDtypeStruct((N_ROWS, EMB), jnp.float32),
    )(gT, mT, eye)


def _sc_gather(idxs, T):
    mesh = plsc.VectorSubcoreMesh(core_axis_name="c", subcore_axis_name="s")
    row_t = jax.ShapeDtypeStruct((B, EMB), jnp.float32)
    buf_t = pltpu.VMEM((CHUNK, EMB), jnp.float32)

    @functools.partial(
        pl.kernel,
        out_type=row_t,
        mesh=mesh,
        scratch_types=[
            pltpu.VMEM((BPW,), jnp.int32),
            buf_t, buf_t,
            pltpu.SemaphoreType.DMA,
            pltpu.SemaphoreType.DMA,
        ],
    )
    def gather_kernel(i_hbm, T_hbm, o_hbm, idx, b0, b1, s0, s1):
        wid = lax.axis_index("s") * NC + lax.axis_index("c")
        base = wid * BPW
        pltpu.sync_copy(i_hbm.at[pl.ds(base, BPW)], idx)

        def fire(c, buf, sem):
            off = c * CHUNK

            @pl.loop(0, CHUNK, step=16)
            def _(r):
                vec = idx[pl.ds(off + r, 16)]
                for j in range(16):
                    pltpu.async_copy(T_hbm.at[pl.ds(vec[j], 1)],
                                     buf.at[pl.ds(r + j, 1)], sem)

        def drain_store(c, buf, sem):
            pltpu.make_async_copy(T_hbm.at[pl.ds(0, CHUNK)], buf, sem).wait()
            pltpu.sync_copy(buf, o_hbm.at[pl.ds(base + c * CHUNK, CHUNK)])

        fire(0, b0, s0)
        for c in range(NCHUNK):
            if c + 1 < NCHUNK:
                fire(c + 1, (b0, b1)[(c + 1) % 2], (s0, s1)[(c + 1) % 2])
            drain_store(c, (b0, b1)[c % 2], (s0, s1)[c % 2])

    return gather_kernel(idxs, T)


def _unpack(x):
    w = lax.bitcast_convert_type(x, jnp.uint32)
    hi = lax.bitcast_convert_type(w & HI_MASK, jnp.float32)
    lo = lax.bitcast_convert_type(w << 16, jnp.float32)
    return hi, lo


def _bf16(x):
    return x.astype(jnp.bfloat16)


def _dotT(w, x):
    return lax.dot_general(w, x, (((1,), (1,)), ((), ())),
                           preferred_element_type=jnp.float32)


def _tc_body(u_ref, v_ref, w1u_ref, w1i_ref, eye_ref, b1_ref, w2_ref,
             b2_ref, w3_ref, b3_ref, o_ref):
    gmf_u, mlp_u = _unpack(u_ref[...])
    gmf_v, mlp_v = _unpack(v_ref[...])
    h1 = _dotT(w1u_ref[...], _bf16(mlp_u)) + _dotT(w1i_ref[...], _bf16(mlp_v))
    h1 = jnp.maximum(h1 + b1_ref[...], 0.0)
    h2 = jnp.dot(w2_ref[...], _bf16(h1), preferred_element_type=jnp.float32)
    h2 = jnp.maximum(h2 + b2_ref[...], 0.0)
    mlp = jnp.dot(w3_ref[...], _bf16(h2),
                  preferred_element_type=jnp.float32)
    pT = _dotT(eye_ref[...], _bf16(gmf_u * gmf_v))
    gmf = jnp.sum(pT, axis=0, keepdims=True)
    o_ref[...] = jax.nn.sigmoid(gmf + mlp + b3_ref[...])[0]


def _tc_dense(gu, gi, W1, b1, W2, b2, W3, b3):
    w1u = W1[:, :EMB].astype(jnp.bfloat16)
    w1i = W1[:, EMB:].astype(jnp.bfloat16)
    w2b = W2.astype(jnp.bfloat16)
    w3b = W3.astype(jnp.bfloat16)
    eye = jnp.eye(EMB, dtype=jnp.bfloat16)
    b1c = b1.reshape(-1, 1)
    b2c = b2.reshape(-1, 1)
    b3c = b3.reshape(1, 1)

    grid = B // TC_BLK
    blk = lambda: pl.BlockSpec((TC_BLK, EMB), lambda i: (i, 0))
    full = lambda a: pl.BlockSpec(a.shape, lambda i: (0,) * a.ndim)
    out = pl.pallas_call(
        _tc_body,
        grid=(grid,),
        in_specs=[blk(), blk(),
                  full(w1u), full(w1i), full(eye), full(b1c),
                  full(w2b), full(b2c), full(w3b), full(b3c)],
        out_specs=pl.BlockSpec((TC_BLK,), lambda i: (i,)),
        out_shape=jax.ShapeDtypeStruct((B,), jnp.float32),
    )(gu, gi, w1u, w1i, eye, b1c, w2b, b2c, w3b, b3c)
    return out


def kernel(users, items, gmf_user_W, gmf_item_W, mlp_user_W, mlp_item_W,
           W1, b1, W2, b2, W3, b3):
    users = users.astype(jnp.int32)
    items = items.astype(jnp.int32)
    U = _tc_prep(gmf_user_W.T, mlp_user_W.T)
    gu = _sc_gather(users, U)
    V = _tc_prep(gmf_item_W.T, mlp_item_W.T)
    gi = _sc_gather(items, V)
    return _tc_dense(gu, gi, W1, b1, W2, b2, W3, b3)
